# Initial kernel scaffold; baseline (speedup 1.0000x reference)
import functools

import jax
import jax.numpy as jnp
from jax import lax
from jax.experimental import pallas as pl
from jax.experimental.pallas import tpu as pltpu
from jax.experimental.pallas import tpu_sc as plsc

VOCAB = 1000000
EMBED = 32
B, F, P = 4096, 26, 20
BF = B * F
NW = 32
WPW = BF // NW
N = 64
C = WPW // N
IDS = N * P
G = 128
NG = IDS // G
HALF = EMBED // 2


def _body(ids_hbm, table_hbm, out_hbm, idx_v, rows_v, out_v, inv_v, sem):
    wid = lax.axis_index("s") * 2 + lax.axis_index("c")
    row0 = wid * WPW
    lane = lax.iota(jnp.int32, 16)

    def chunk_body(c, carry):
        base = (row0 + c * N) * P
        pltpu.sync_copy(ids_hbm.at[pl.ds(base, IDS)], idx_v)
        copies = []
        for g in range(NG):
            copies.append(pltpu.async_copy(
                table_hbm.at[idx_v.at[pl.ds(g * G, G)]],
                rows_v.at[pl.ds(g * G, G)], sem))
        for cp in copies:
            cp.wait()

        for jj in range(N // 16):
            cnt = jnp.zeros((16,), jnp.float32)
            pos0 = lane * P + jj * 16 * P
            for p in range(P):
                g = plsc.load_gather(idx_v, [pos0 + p])
                cnt = cnt + jnp.where(g != 0, 1.0, 0.0)
            inv = 1.0 / jnp.maximum(cnt, 1.0)
            inv_v[pl.ds(jj * 16, 16)] = inv

        def sum_body(n, carry2):
            inv = inv_v[n]
            r = n * P
            for h in range(2):
                acc = rows_v[r, pl.ds(h * HALF, HALF)]
                for p in range(1, P):
                    acc = acc + rows_v[r + p, pl.ds(h * HALF, HALF)]
                out_v[n, pl.ds(h * HALF, HALF)] = acc * inv
            return carry2

        lax.fori_loop(0, N, sum_body, 0)
        pltpu.sync_copy(out_v, out_hbm.at[pl.ds(row0 + c * N, N)])
        return carry

    lax.fori_loop(0, C, chunk_body, 0)


def kernel(path_ids, table):
    ids_flat = path_ids.reshape(-1).astype(jnp.int32)
    mesh = plsc.VectorSubcoreMesh(core_axis_name="c", subcore_axis_name="s")
    run = functools.partial(
        pl.kernel,
        out_type=jax.ShapeDtypeStruct((BF, EMBED), jnp.float32),
        mesh=mesh,
        scratch_types=[
            pltpu.VMEM((IDS,), jnp.int32),
            pltpu.VMEM((IDS, EMBED), jnp.float32),
            pltpu.VMEM((N, EMBED), jnp.float32),
            pltpu.VMEM((N,), jnp.float32),
            pltpu.SemaphoreType.DMA,
        ],
    )(_body)
    out = run(ids_flat, table)
    return out.reshape(B, F, EMBED)


# speedup vs baseline: 5.1505x; 5.1505x over previous
import functools

import jax
import jax.numpy as jnp
from jax import lax
from jax.experimental import pallas as pl
from jax.experimental.pallas import tpu as pltpu
from jax.experimental.pallas import tpu_sc as plsc

VOCAB = 1000000
EMBED = 32
B, F, P = 4096, 26, 20
BF = B * F
NW = 32
WPW = BF // NW
N = 64
C = WPW // N
IDS = N * P
G = 128
NG = IDS // G
HALF = EMBED // 2


def _body(ids_hbm, table_hbm, out_hbm, idx_v, rows_v, out_v, sem):
    wid = lax.axis_index("s") * 2 + lax.axis_index("c")
    row0 = wid * WPW

    def chunk_body(c, carry):
        base = (row0 + c * N) * P
        pltpu.sync_copy(ids_hbm.at[pl.ds(base, IDS)], idx_v)
        copies = []
        for g in range(NG):
            copies.append(pltpu.async_copy(
                table_hbm.at[idx_v.at[pl.ds(g * G, G)]],
                rows_v.at[pl.ds(g * G, G)], sem))
        for cp in copies:
            cp.wait()

        def blk_body(jj, carry2):
            n0 = jj * 16
            cnt = jnp.zeros((16,), jnp.float32)
            for p in range(P):
                g = idx_v[pl.ds(p * N + n0, 16)]
                cnt = cnt + jnp.where(g != 0, 1.0, 0.0)
            inv16 = 1.0 / jnp.maximum(cnt, 1.0)
            for l in range(16):
                inv_s = inv16[l]
                for h in range(2):
                    acc = rows_v[n0 + l, pl.ds(h * HALF, HALF)]
                    for p in range(1, P):
                        acc = acc + rows_v[p * N + n0 + l, pl.ds(h * HALF, HALF)]
                    out_v[n0 + l, pl.ds(h * HALF, HALF)] = acc * inv_s
            return carry2

        lax.fori_loop(0, N // 16, blk_body, 0)
        pltpu.sync_copy(out_v, out_hbm.at[pl.ds(row0 + c * N, N)])
        return carry

    lax.fori_loop(0, C, chunk_body, 0)


def kernel(path_ids, table):
    ids_flat = (path_ids.reshape(NW, C, N, P)
                .transpose(0, 1, 3, 2)
                .reshape(-1)
                .astype(jnp.int32))
    mesh = plsc.VectorSubcoreMesh(core_axis_name="c", subcore_axis_name="s")
    run = functools.partial(
        pl.kernel,
        out_type=jax.ShapeDtypeStruct((BF, EMBED), jnp.float32),
        mesh=mesh,
        compiler_params=pltpu.CompilerParams(use_tc_tiling_on_sc=False),
        scratch_types=[
            pltpu.VMEM((IDS,), jnp.int32),
            pltpu.VMEM((IDS, EMBED), jnp.float32),
            pltpu.VMEM((N, EMBED), jnp.float32),
            pltpu.SemaphoreType.DMA,
        ],
    )(_body)
    out = run(ids_flat, table)
    return out.reshape(B, F, EMBED)


# speedup vs baseline: 5.6438x; 1.0958x over previous
import functools

import jax
import jax.numpy as jnp
from jax import lax
from jax.experimental import pallas as pl
from jax.experimental.pallas import tpu as pltpu
from jax.experimental.pallas import tpu_sc as plsc

VOCAB = 1000000
EMBED = 32
B, F, P = 4096, 26, 20
BF = B * F
NW = 32
WPW = BF // NW
N = 64
C = WPW // N
IDS = N * P
G = 128
NG = IDS // G
HALF = EMBED // 2


def _body(ids_hbm, table_hbm, out_hbm,
          idx0, idx1, rows0, rows1, out_v, sem0, sem1):
    wid = lax.axis_index("s") * 2 + lax.axis_index("c")
    row0 = wid * WPW

    def issue(c, idx_v, rows_v, sem):
        base = (row0 + c * N) * P
        pltpu.sync_copy(ids_hbm.at[pl.ds(base, IDS)], idx_v)
        for g in range(NG):
            pltpu.async_copy(
                table_hbm.at[idx_v.at[pl.ds(g * G, G)]],
                rows_v.at[pl.ds(g * G, G)], sem)

    def drain(idx_v, rows_v, sem):
        for g in range(NG):
            pltpu.make_async_copy(
                table_hbm.at[idx_v.at[pl.ds(g * G, G)]],
                rows_v.at[pl.ds(g * G, G)], sem).wait()

    def compute(c, idx_v, rows_v):
        def blk_body(jj, carry2):
            n0 = jj * 16
            cnt = jnp.zeros((16,), jnp.float32)
            for p in range(P):
                g = idx_v[pl.ds(p * N + n0, 16)]
                cnt = cnt + jnp.where(g != 0, 1.0, 0.0)
            inv16 = 1.0 / jnp.maximum(cnt, 1.0)
            for l in range(16):
                inv_s = inv16[l]
                for h in range(2):
                    acc = rows_v[n0 + l, pl.ds(h * HALF, HALF)]
                    for p in range(1, P):
                        acc = acc + rows_v[p * N + n0 + l, pl.ds(h * HALF, HALF)]
                    out_v[n0 + l, pl.ds(h * HALF, HALF)] = acc * inv_s
            return carry2

        lax.fori_loop(0, N // 16, blk_body, 0)
        pltpu.sync_copy(out_v, out_hbm.at[pl.ds(row0 + c * N, N)])

    issue(0, idx0, rows0, sem0)

    def pair_body(i, carry):
        c0 = i * 2
        issue(c0 + 1, idx1, rows1, sem1)
        drain(idx0, rows0, sem0)
        compute(c0, idx0, rows0)

        @pl.when(c0 + 2 < C)
        def _():
            issue(c0 + 2, idx0, rows0, sem0)

        drain(idx1, rows1, sem1)
        compute(c0 + 1, idx1, rows1)
        return carry

    lax.fori_loop(0, C // 2, pair_body, 0)


def kernel(path_ids, table):
    ids_flat = (path_ids.reshape(NW, C, N, P)
                .transpose(0, 1, 3, 2)
                .reshape(-1)
                .astype(jnp.int32))
    mesh = plsc.VectorSubcoreMesh(core_axis_name="c", subcore_axis_name="s")
    run = functools.partial(
        pl.kernel,
        out_type=jax.ShapeDtypeStruct((BF, EMBED), jnp.float32),
        mesh=mesh,
        compiler_params=pltpu.CompilerParams(use_tc_tiling_on_sc=False),
        scratch_types=[
            pltpu.VMEM((IDS,), jnp.int32),
            pltpu.VMEM((IDS,), jnp.int32),
            pltpu.VMEM((IDS, EMBED), jnp.float32),
            pltpu.VMEM((IDS, EMBED), jnp.float32),
            pltpu.VMEM((N, EMBED), jnp.float32),
            pltpu.SemaphoreType.DMA,
            pltpu.SemaphoreType.DMA,
        ],
    )(_body)
    out = run(ids_flat, table)
    return out.reshape(B, F, EMBED)
